# Initial kernel scaffold; baseline (speedup 1.0000x reference)
import jax
import jax.numpy as jnp
from jax.experimental import pallas as pl


def kernel(inputs, dense_emb, wide_emb, W1, b1, W2, b2, W3, b3, Wp, bp):
    raise NotImplementedError("write your pallas kernel here")


# speedup vs baseline: 3.5542x; 3.5542x over previous
import functools

import jax
import jax.numpy as jnp
from jax import lax
from jax.experimental import pallas as pl
from jax.experimental.pallas import tpu as pltpu
from jax.experimental.pallas import tpu_sc as plsc

_V = 1000000
_D = 16
_F = 26
_H = 100
_B = 4096

_NC = 2
_NS = 16
_NW = _NC * _NS
_IPW = _B * _F // _NW
_CH = 128
_NCH = _IPW // _CH
_ROWW = 128
_NBUF = 2


@functools.lru_cache(maxsize=None)
def _make_sc_gather():
    mesh = plsc.VectorSubcoreMesh(core_axis_name="c", subcore_axis_name="s")

    @functools.partial(
        pl.kernel,
        out_type=(
            jax.ShapeDtypeStruct((_NW, _NCH, _CH * _D), jnp.float32),
            jax.ShapeDtypeStruct((_NW, _NCH, _CH), jnp.float32),
        ),
        mesh=mesh,
        compiler_params=pltpu.CompilerParams(needs_layout_passes=False),
        scratch_types=[
            pltpu.VMEM((_NCH, _CH), jnp.int32),
            pltpu.VMEM((_NCH, _CH), jnp.int32),
            pltpu.VMEM((_NCH, _CH), jnp.int32),
            pltpu.VMEM((_NBUF, _CH, _ROWW), jnp.float32),
            pltpu.VMEM((_NCH, _CH * _D), jnp.float32),
            pltpu.VMEM((_NCH, _CH), jnp.float32),
            pltpu.SemaphoreType.DMA,
            pltpu.SemaphoreType.DMA,
            pltpu.SemaphoreType.DMA,
        ],
    )
    def sc_gather(rid_hbm, off_hbm, idx_hbm, table_hbm, wide_hbm,
                  emb_out, wide_out,
                  rid_v, off_v, idx_v, raw_v, ext_v, wvals_v,
                  sem_a, sem_b, sem_w):
        wid = lax.axis_index("s") * _NC + lax.axis_index("c")
        pltpu.sync_copy(rid_hbm.at[wid], rid_v)
        pltpu.sync_copy(off_hbm.at[wid], off_v)
        pltpu.sync_copy(idx_hbm.at[wid], idx_v)

        wide_copies = [
            pltpu.async_copy(wide_hbm.at[idx_v.at[j]], wvals_v.at[j], sem_w)
            for j in range(_NCH)
        ]

        sems = [sem_a, sem_b]
        lanes = lax.iota(jnp.int32, 16)

        for b in range(_NBUF):
            pltpu.async_copy(table_hbm.at[rid_v.at[b]], raw_v.at[b], sems[b])

        def extract(j, b):
            raw = raw_v.at[b]
            offs = off_v.at[j]
            jvec = jnp.full((16,), j, jnp.int32)

            @pl.loop(0, _CH // 16)
            def _groups(k):
                rows16 = k * 16 + lanes
                off16 = offs[pl.ds(k * 16, 16)]
                dst_base = rows16 * _D
                for d in range(_D):
                    vals = plsc.load_gather(raw, [rows16, off16 + d])
                    plsc.store_scatter(ext_v, [jvec, dst_base + d], vals)

        @pl.loop(0, _NCH // _NBUF)
        def _chunks(g):
            for b in range(_NBUF):
                j = g * _NBUF + b
                pltpu.make_async_copy(
                    table_hbm.at[pl.ds(0, _CH)], raw_v.at[b], sems[b]
                ).wait()
                extract(j, b)
                jn = j + _NBUF

                @pl.when(jn < _NCH)
                def _():
                    pltpu.async_copy(
                        table_hbm.at[rid_v.at[jn]], raw_v.at[b], sems[b]
                    )

        for c in wide_copies:
            c.wait()
        pltpu.sync_copy(ext_v, emb_out.at[wid])
        pltpu.sync_copy(wvals_v, wide_out.at[wid])

    return sc_gather


_BB = 512


def _mlp_body(x_ref, wv_ref, w1_ref, b1_ref, w2_ref, b2_ref, w3_ref, b3_ref,
              wp_ref, bp_ref, o_ref):
    x = x_ref[...]
    h = jnp.maximum(jnp.dot(x, w1_ref[...], preferred_element_type=jnp.float32)
                    + b1_ref[...], 0.0)
    h = jnp.maximum(jnp.dot(h, w2_ref[...], preferred_element_type=jnp.float32)
                    + b2_ref[...], 0.0)
    h = jnp.maximum(jnp.dot(h, w3_ref[...], preferred_element_type=jnp.float32)
                    + b3_ref[...], 0.0)
    wide = jnp.sum(wv_ref[...], axis=1, keepdims=True)
    logits = (jnp.dot(h, wp_ref[...], preferred_element_type=jnp.float32)
              + bp_ref[...] + wide)
    o_ref[...] = jax.nn.sigmoid(logits)


@jax.jit
def _mlp(emb, wv, W1, b1, W2, b2, W3, b3, Wp, bp):
    grid = (_B // _BB,)
    return pl.pallas_call(
        _mlp_body,
        grid=grid,
        in_specs=[
            pl.BlockSpec((_BB, _F * _D), lambda i: (i, 0)),
            pl.BlockSpec((_BB, _F), lambda i: (i, 0)),
            pl.BlockSpec((_F * _D, _H), lambda i: (0, 0)),
            pl.BlockSpec((1, _H), lambda i: (0, 0)),
            pl.BlockSpec((_H, _H), lambda i: (0, 0)),
            pl.BlockSpec((1, _H), lambda i: (0, 0)),
            pl.BlockSpec((_H, _H), lambda i: (0, 0)),
            pl.BlockSpec((1, _H), lambda i: (0, 0)),
            pl.BlockSpec((_H, 1), lambda i: (0, 0)),
            pl.BlockSpec((1, 1), lambda i: (0, 0)),
        ],
        out_specs=pl.BlockSpec((_BB, 1), lambda i: (i, 0)),
        out_shape=jax.ShapeDtypeStruct((_B, 1), jnp.float32),
    )(emb, wv, W1, b1, W2, b2, W3, b3, Wp, bp)


def kernel(inputs, dense_emb, wide_emb, W1, b1, W2, b2, W3, b3, Wp, bp):
    idx = inputs.reshape(_NW, _NCH, _CH)
    rid = idx // 8
    off = (idx % 8) * _D
    table = dense_emb.reshape(_V * _D // _ROWW, _ROWW)
    emb3, wv3 = _make_sc_gather()(rid, off, idx, table, wide_emb.reshape(-1))
    emb = emb3.reshape(_B, _F * _D)
    wv = wv3.reshape(_B, _F)
    return _mlp(emb, wv, W1, b1.reshape(1, _H), W2, b2.reshape(1, _H),
                W3, b3.reshape(1, _H), Wp, bp.reshape(1, 1))
